# baseline (speedup 1.0000x reference)
import functools

import numpy as np
import jax
import jax.numpy as jnp
from jax import lax
from jax.experimental import pallas as pl
from jax.experimental.pallas import tpu as pltpu
from jax.experimental.pallas import tpu_sc as plsc

B, C, H, W = 4, 32, 160, 160
K = 32
G0 = 8
OUT_H = OUT_W = 79
P = OUT_H * OUT_W
HQ = WQ = 80
NPL = C * 4
HHR = 41
HALF = HHR * WQ
R_SPLIT = 40

_OPS_M = np.array([
    [0, 0, 0, 0],
    [0, 0, 0, 1],
    [0, 1, 0, -1],
    [0, 1, 0, 0],
    [0, 0, 1, -1],
    [0, 0, 1, 0],
    [0, 1, 1, -2],
    [0, 1, 1, -1],
    [1, -1, -1, 1],
    [1, -1, -1, 2],
    [1, 0, -1, 0],
    [1, 0, -1, 1],
    [1, -1, 0, 0],
    [1, -1, 0, 1],
    [1, 0, 0, -1],
    [1, 0, 0, 0],
], np.float32)

_COL_BASES = (0, 16, 32, 48, 63)


def _sc_call(xph, meta_p, meta_s, coef):
    mesh = plsc.VectorSubcoreMesh(
        core_axis_name="c", subcore_axis_name="s", num_cores=2, num_subcores=16)

    @functools.partial(
        pl.kernel,
        mesh=mesh,
        compiler_params=pltpu.CompilerParams(
            use_tc_tiling_on_sc=False, needs_layout_passes=False),
        out_type=jax.ShapeDtypeStruct((B, K, P), jnp.float32),
        scratch_types=[
            pltpu.VMEM((16, HALF), jnp.float32),
            pltpu.VMEM((16, HALF), jnp.float32),
            pltpu.VMEM((P,), jnp.float32),
            pltpu.VMEM((P,), jnp.float32),
            pltpu.VMEM((16,), jnp.int32),
            pltpu.VMEM((16,), jnp.int32),
            pltpu.VMEM((64,), jnp.float32),
            pltpu.SemaphoreType.DMA,
            pltpu.SemaphoreType.DMA,
            pltpu.SemaphoreType.DMA,
            pltpu.SemaphoreType.DMA,
        ],
    )
    def body(xph_hbm, mp_hbm, ms_hbm, cf_hbm, out_hbm, pln0, pln1, ob0, ob1,
             mpv, msv, cfv, si0, si1, so0, so1):
        k = lax.axis_index("s") * 2 + lax.axis_index("c")
        pltpu.sync_copy(mp_hbm.at[k], mpv)
        pltpu.sync_copy(ms_hbm.at[k], msv)
        pltpu.sync_copy(cf_hbm.at[k], cfv)

        rows2 = mpv[...] * 2
        svec = msv[...]
        shifts = [svec[g] for g in range(16)]
        cvecs = [cfv[pl.ds(16 * j, 16)] for j in range(4)]
        cf = [[cvecs[j][n] for j in range(4)] for n in range(15)]

        def comb(a_, b_, cn):
            return cn[0] + cn[1] * a_ + b_ * (cn[2] + cn[3] * a_)

        plns = (pln0, pln1)
        sis = (si0, si1)
        obs = (ob0, ob1)
        sos = (so0, so1)
        chunks = [(b, h) for b in range(B) for h in range(2)]

        def start_gather(i):
            b, h = chunks[i]
            idx = rows2 + np.int32(b * 2 * NPL + h)
            return pltpu.async_copy(xph_hbm.at[idx], plns[i % 2], sis[i % 2])

        out_handles = [None] * B
        in_handles = [None] * 8
        in_handles[0] = start_gather(0)
        for i, (b, h) in enumerate(chunks):
            if i + 1 < len(chunks):
                in_handles[i + 1] = start_gather(i + 1)
            in_handles[i].wait()
            if h == 0 and b >= 2:
                out_handles[b - 2].wait()
            planes = plns[i % 2]
            outb = obs[b % 2]

            lo, hi = (0, R_SPLIT) if h == 0 else (R_SPLIT, OUT_H)
            rb_off = 0 if h == 0 else -(R_SPLIT - 1) * WQ

            @plsc.parallel_loop(lo, hi, unroll=2)
            def row_body(r):
                rb = r * WQ + rb_off
                ob = r * OUT_W
                for cb in _COL_BASES:
                    vals = [planes[g, pl.ds(rb + cb + shifts[g], 16)]
                            for g in range(16)]
                    t = [comb(vals[g], vals[8 + g], cf[g]) for g in range(G0)]
                    u = [comb(t[2 * j], t[2 * j + 1], cf[8 + j]) for j in range(4)]
                    v = [comb(u[2 * j], u[2 * j + 1], cf[12 + j]) for j in range(2)]
                    outb[pl.ds(ob + cb, 16)] = comb(v[0], v[1], cf[14])

            if h == 1:
                out_handles[b] = pltpu.async_copy(
                    outb, out_hbm.at[b, k], sos[b % 2])

        out_handles[B - 2].wait()
        out_handles[B - 1].wait()

    return body(xph, meta_p, meta_s, coef)


def kernel(x, idx_a, idx_b, w0, w1, w2, w3):
    xp4 = (x.reshape(B, C, HQ, 2, WQ, 2)
            .transpose(0, 1, 3, 5, 2, 4)
            .reshape(B * NPL, HQ, WQ))
    xph = jnp.stack([xp4[:, 0:HHR], xp4[:, HQ - HHR:HQ]],
                    axis=1).reshape(B * NPL * 2, HALF)

    def side_meta(idx):
        dh, dw, ch = idx[0, :, :, 0], idx[0, :, :, 1], idx[0, :, :, 2]
        poff = ch * 4 + (dh % 2) * 2 + (dw % 2)
        shift = (dh // 2) * WQ + (dw // 2)
        return poff.astype(jnp.int32), shift.astype(jnp.int32)

    pa, sa = side_meta(idx_a)
    pb, sb = side_meta(idx_b)
    meta_p = jnp.concatenate([pa, pb], axis=1)
    meta_s = jnp.concatenate([sa, sb], axis=1)

    M = jnp.asarray(_OPS_M)
    coefs = [jnp.einsum('gki,ij->kjg', jax.nn.softmax(w, axis=-1), M,
                        precision=lax.Precision.HIGHEST)
             for w in (w0, w1, w2, w3)]
    coef = jnp.concatenate(coefs, axis=-1)
    coef = jnp.pad(coef, ((0, 0), (0, 0), (0, 1))).reshape(K, 64)

    out = _sc_call(xph, meta_p, meta_s, coef)
    return out.reshape(B, K, OUT_H, OUT_W)


# speedup vs baseline: 1.2388x; 1.2388x over previous
import functools

import numpy as np
import jax
import jax.numpy as jnp
from jax import lax
from jax.experimental import pallas as pl
from jax.experimental.pallas import tpu as pltpu
from jax.experimental.pallas import tpu_sc as plsc

B, C, H, W = 4, 32, 160, 160
K = 32
G0 = 8
OUT_H = OUT_W = 79
P = OUT_H * OUT_W
HQ = WQ = 80
NPL = C * 4
HHR = 41
NROW = 16 * HHR
HALF = HHR * WQ
NCH = 2 * B
R_SPLIT = 40

_OPS_M = np.array([
    [0, 0, 0, 0],
    [0, 0, 0, 1],
    [0, 1, 0, -1],
    [0, 1, 0, 0],
    [0, 0, 1, -1],
    [0, 0, 1, 0],
    [0, 1, 1, -2],
    [0, 1, 1, -1],
    [1, -1, -1, 1],
    [1, -1, -1, 2],
    [1, 0, -1, 0],
    [1, 0, -1, 1],
    [1, -1, 0, 0],
    [1, -1, 0, 1],
    [1, 0, 0, -1],
    [1, 0, 0, 0],
], np.float32)

_COL_BASES = (0, 16, 32, 48, 63)


def _sc_call(xp, ridx, meta_s, coef):
    mesh = plsc.VectorSubcoreMesh(
        core_axis_name="c", subcore_axis_name="s", num_cores=2, num_subcores=16)

    @functools.partial(
        pl.kernel,
        mesh=mesh,
        compiler_params=pltpu.CompilerParams(
            use_tc_tiling_on_sc=False, needs_layout_passes=False),
        out_type=jax.ShapeDtypeStruct((B, K, P), jnp.float32),
        scratch_types=[
            pltpu.VMEM((NROW, WQ), jnp.float32),
            pltpu.VMEM((NROW, WQ), jnp.float32),
            pltpu.VMEM((P,), jnp.float32),
            pltpu.VMEM((P,), jnp.float32),
            pltpu.VMEM((NCH, NROW), jnp.int32),
            pltpu.VMEM((32,), jnp.int32),
            pltpu.VMEM((64,), jnp.float32),
            pltpu.SemaphoreType.DMA,
            pltpu.SemaphoreType.DMA,
            pltpu.SemaphoreType.DMA,
            pltpu.SemaphoreType.DMA,
        ],
    )
    def body(xp_hbm, ri_hbm, ms_hbm, cf_hbm, out_hbm, pln0, pln1, ob0, ob1,
             idxs, msv, cfv, si0, si1, so0, so1):
        k = lax.axis_index("s") * 2 + lax.axis_index("c")
        pltpu.sync_copy(ri_hbm.at[k], idxs)
        pltpu.sync_copy(ms_hbm.at[k], msv)
        pltpu.sync_copy(cf_hbm.at[k], cfv)

        gvec = msv[pl.ds(0, 16)]
        dvec = msv[pl.ds(16, 16)]
        grows = [gvec[g] for g in range(16)]
        dwqs = [dvec[g] for g in range(16)]
        cvecs = [cfv[pl.ds(16 * j, 16)] for j in range(4)]
        cf = [[cvecs[j][n] for j in range(4)] for n in range(15)]

        def comb(a_, b_, cn):
            return cn[0] + cn[1] * a_ + b_ * (cn[2] + cn[3] * a_)

        plns = (pln0, pln1)
        sis = (si0, si1)
        obs = (ob0, ob1)
        sos = (so0, so1)
        chunks = [(b, h) for b in range(B) for h in range(2)]

        def start_gather(i):
            return pltpu.async_copy(
                xp_hbm.at[idxs.at[i]], plns[i % 2], sis[i % 2])

        out_handles = [None] * B
        in_handles = [None] * NCH
        in_handles[0] = start_gather(0)
        for i, (b, h) in enumerate(chunks):
            if i + 1 < NCH:
                in_handles[i + 1] = start_gather(i + 1)
            in_handles[i].wait()
            if h == 0 and b >= 2:
                out_handles[b - 2].wait()
            planes = plns[i % 2]
            outb = obs[b % 2]

            lo, hi = (0, R_SPLIT) if h == 0 else (R_SPLIT, OUT_H)
            ir_off = 0 if h == 0 else -(R_SPLIT - 1)

            @plsc.parallel_loop(lo, hi, unroll=2)
            def row_body(r):
                ir = r + ir_off
                ob = r * OUT_W
                for cb in _COL_BASES:
                    vals = [planes[grows[g] + ir, pl.ds(cb + dwqs[g], 16)]
                            for g in range(16)]
                    t = [comb(vals[g], vals[8 + g], cf[g]) for g in range(G0)]
                    u = [comb(t[2 * j], t[2 * j + 1], cf[8 + j]) for j in range(4)]
                    v = [comb(u[2 * j], u[2 * j + 1], cf[12 + j]) for j in range(2)]
                    outb[pl.ds(ob + cb, 16)] = comb(v[0], v[1], cf[14])

            if h == 1:
                out_handles[b] = pltpu.async_copy(
                    outb, out_hbm.at[b, k], sos[b % 2])

        out_handles[B - 2].wait()
        out_handles[B - 1].wait()

    return body(xp, ridx, meta_s, coef)


def kernel(x, idx_a, idx_b, w0, w1, w2, w3):
    xp = (x.reshape(B, C, HQ, 2, WQ, 2)
           .transpose(0, 1, 3, 5, 2, 4)
           .reshape(B * NPL * HQ, WQ))

    def side_meta(idx):
        dh, dw, ch = idx[0, :, :, 0], idx[0, :, :, 1], idx[0, :, :, 2]
        poff = ch * 4 + (dh % 2) * 2 + (dw % 2)
        shift = (dh // 2) * WQ + (dw // 2)
        return poff.astype(jnp.int32), shift.astype(jnp.int32)

    pa, sa = side_meta(idx_a)
    pb, sb = side_meta(idx_b)
    pid = jnp.concatenate([pa, pb], axis=1)
    shift = jnp.concatenate([sa, sb], axis=1)
    grow = jnp.arange(16, dtype=jnp.int32)[None, :] * HHR + shift // WQ
    meta_s = jnp.concatenate([grow, shift % WQ], axis=1)

    b_off = (jnp.arange(B, dtype=jnp.int32) * (NPL * HQ))[None, :, None, None, None]
    h_off = (jnp.arange(2, dtype=jnp.int32) * (R_SPLIT - 1))[None, None, :, None, None]
    i_off = jnp.arange(HHR, dtype=jnp.int32)[None, None, None, None, :]
    ridx = (pid[:, None, None, :, None] * HQ + b_off + h_off + i_off)
    ridx = ridx.reshape(K, NCH, NROW)

    M = jnp.asarray(_OPS_M)
    coefs = [jnp.einsum('gki,ij->kjg', jax.nn.softmax(w, axis=-1), M,
                        precision=lax.Precision.HIGHEST)
             for w in (w0, w1, w2, w3)]
    coef = jnp.concatenate(coefs, axis=-1)
    coef = jnp.pad(coef, ((0, 0), (0, 0), (0, 1))).reshape(K, 64)

    out = _sc_call(xp, ridx, meta_s, coef)
    return out.reshape(B, K, OUT_H, OUT_W)
